# Initial kernel scaffold; baseline (speedup 1.0000x reference)
import jax
import jax.numpy as jnp
from jax.experimental import pallas as pl


def kernel(x, token_table, pos_table):
    raise NotImplementedError("write your pallas kernel here")


# speedup vs baseline: 2.7465x; 2.7465x over previous
import functools

import jax
import jax.numpy as jnp
from jax import lax
from jax.experimental import pallas as pl
from jax.experimental.pallas import tpu as pltpu
from jax.experimental.pallas import tpu_sc as plsc

D = 64
T = 200
NC = 2
NS = 16
NW = NC * NS

SEQ_PER_CHUNK = 2
CHUNK = SEQ_PER_CHUNK * T
GU = 50
NG = CHUNK // GU


@functools.lru_cache(maxsize=None)
def _make_kernel(n_rows: int):
    rows_per_w = n_rows // NW
    n_chunks = rows_per_w // CHUNK
    mesh = plsc.VectorSubcoreMesh(core_axis_name="c", subcore_axis_name="s")

    @functools.partial(
        pl.kernel,
        mesh=mesh,
        compiler_params=pltpu.CompilerParams(use_tc_tiling_on_sc=False),
        out_type=jax.ShapeDtypeStruct((n_rows, D), jnp.float32),
        scratch_types=[
            pltpu.VMEM((NG, GU), jnp.int32),
            pltpu.VMEM((CHUNK, D), jnp.float32),
            pltpu.VMEM((CHUNK, D), jnp.float32),
            pltpu.SemaphoreType.DMA,
        ],
    )
    def k(table_hbm, xr_hbm, pos_hbm, out_hbm, idx_v, rows_v, pos_v, sem):
        wid = lax.axis_index("s") * NC + lax.axis_index("c")
        for s in range(SEQ_PER_CHUNK):
            pltpu.sync_copy(pos_hbm, pos_v.at[pl.ds(s * T, T)])

        def chunk_body(c, carry):
            base = pl.multiple_of(wid * rows_per_w + c * CHUNK, CHUNK)
            pltpu.sync_copy(
                xr_hbm.at[pl.ds(pl.multiple_of(base // GU, NG), NG)], idx_v)
            copies = [
                pltpu.async_copy(
                    table_hbm.at[idx_v.at[j]],
                    rows_v.at[pl.ds(j * GU, GU)],
                    sem,
                )
                for j in range(NG)
            ]
            for cp in copies:
                cp.wait()

            def add_body(r, carry2):
                for d in range(D // 16):
                    vec = pos_v[r, pl.ds(d * 16, 16)]
                    plsc.addupdate(rows_v.at[r, pl.ds(d * 16, 16)], vec)
                return carry2

            lax.fori_loop(0, CHUNK, add_body, 0)
            pltpu.sync_copy(rows_v, out_hbm.at[pl.ds(base, CHUNK)])
            return carry

        lax.fori_loop(0, n_chunks, chunk_body, 0)

    return k


def kernel(x, token_table, pos_table):
    b, t = x.shape
    xr = x.reshape(-1, GU).astype(jnp.int32)
    out = _make_kernel(b * t)(token_table, xr, pos_table)
    return out.reshape(b, t, D)


# speedup vs baseline: 3.0710x; 1.1182x over previous
import functools

import jax
import jax.numpy as jnp
from jax import lax
from jax.experimental import pallas as pl
from jax.experimental.pallas import tpu as pltpu
from jax.experimental.pallas import tpu_sc as plsc

D = 64
T = 200
NC = 2
NS = 16
NW = NC * NS

SEQ_PER_CHUNK = 2
CHUNK = SEQ_PER_CHUNK * T
GU = 50
NG = CHUNK // GU


@functools.lru_cache(maxsize=None)
def _make_kernel(n_rows: int):
    rows_per_w = n_rows // NW
    n_chunks = rows_per_w // CHUNK
    n_pairs = n_chunks // 2
    idx_rows = rows_per_w // GU
    mesh = plsc.VectorSubcoreMesh(core_axis_name="c", subcore_axis_name="s")

    @functools.partial(
        pl.kernel,
        mesh=mesh,
        compiler_params=pltpu.CompilerParams(use_tc_tiling_on_sc=False),
        out_type=jax.ShapeDtypeStruct((n_rows, D), jnp.float32),
        scratch_types=[
            pltpu.VMEM((idx_rows, GU), jnp.int32),
            pltpu.VMEM((CHUNK, D), jnp.float32),
            pltpu.VMEM((CHUNK, D), jnp.float32),
            pltpu.VMEM((CHUNK, D), jnp.float32),
            pltpu.SemaphoreType.DMA,
            pltpu.SemaphoreType.DMA,
        ],
    )
    def k(table_hbm, xr_hbm, pos_hbm, out_hbm,
          idx_v, rows_v0, rows_v1, pos_v, sem0, sem1):
        wid = lax.axis_index("s") * NC + lax.axis_index("c")
        bufs = ((rows_v0, sem0), (rows_v1, sem1))

        pltpu.sync_copy(
            xr_hbm.at[pl.ds(pl.multiple_of(wid * idx_rows, 8), idx_rows)],
            idx_v)
        for s in range(SEQ_PER_CHUNK):
            pltpu.sync_copy(pos_hbm, pos_v.at[pl.ds(s * T, T)])

        def fire(c, buf, sem):
            return [
                pltpu.async_copy(
                    table_hbm.at[idx_v.at[c * NG + j]],
                    buf.at[pl.ds(j * GU, GU)],
                    sem,
                )
                for j in range(NG)
            ]

        def drain(buf, sem):
            for j in range(NG):
                pltpu.make_async_copy(
                    table_hbm.at[idx_v.at[j]],
                    buf.at[pl.ds(j * GU, GU)],
                    sem,
                ).wait()

        def compute_and_writeback(c, buf):
            @plsc.parallel_loop(0, CHUNK, unroll=8)
            def _(r):
                for d in range(D // 16):
                    plsc.addupdate(buf.at[r, pl.ds(d * 16, 16)],
                                   pos_v[r, pl.ds(d * 16, 16)])
            base = pl.multiple_of(wid * rows_per_w + c * CHUNK, CHUNK)
            pltpu.sync_copy(buf, out_hbm.at[pl.ds(base, CHUNK)])

        fire(0, rows_v0, sem0)

        def pair_body(g, carry):
            c = g * 2
            fire(c + 1, rows_v1, sem1)
            drain(rows_v0, sem0)
            compute_and_writeback(c, rows_v0)
            @pl.when(g < n_pairs - 1)
            def _():
                fire(c + 2, rows_v0, sem0)
            drain(rows_v1, sem1)
            compute_and_writeback(c + 1, rows_v1)
            return carry

        lax.fori_loop(0, n_pairs, pair_body, 0)

    return k


def kernel(x, token_table, pos_table):
    b, t = x.shape
    xr = x.reshape(-1, GU).astype(jnp.int32)
    out = _make_kernel(b * t)(token_table, xr, pos_table)
    return out.reshape(b, t, D)


# speedup vs baseline: 4.0035x; 1.3037x over previous
import functools

import jax
import jax.numpy as jnp
from jax import lax
from jax.experimental import pallas as pl
from jax.experimental.pallas import tpu as pltpu
from jax.experimental.pallas import tpu_sc as plsc

D = 64
T = 200
B = 1024
V = 100000
NC = 2
NS = 16
NW = NC * NS

TSLAB = 8


@functools.lru_cache(maxsize=None)
def _make_kernel():
    mesh = plsc.VectorSubcoreMesh(core_axis_name="c", subcore_axis_name="s")

    @functools.partial(
        pl.kernel,
        mesh=mesh,
        compiler_params=pltpu.CompilerParams(
            use_tc_tiling_on_sc=False, needs_layout_passes=False),
        out_type=jax.ShapeDtypeStruct((T, D, B), jnp.float32),
        scratch_types=[
            pltpu.VMEM((V,), jnp.float32),
            pltpu.VMEM((TSLAB, B), jnp.int32),
            pltpu.VMEM((TSLAB, B), jnp.float32),
            pltpu.VMEM((T + 16,), jnp.float32),
            pltpu.VMEM_SHARED((T, B), jnp.int32),
        ],
    )
    def k(tab_hbm, xt_hbm, pos_hbm, out_hbm, row_v, xs_v, out_v, pos_v, x_sp):
        cid = lax.axis_index("c")
        sid = lax.axis_index("s")
        wid = sid * NC + cid

        @pl.when(sid == 0)
        def _():
            pltpu.sync_copy(xt_hbm, x_sp)
        plsc.subcore_barrier()

        for p in range(D // NW):
            j = p * NW + wid
            pltpu.sync_copy(tab_hbm.at[j], row_v)
            pltpu.sync_copy(pos_hbm.at[j], pos_v.at[pl.ds(0, T)])

            def slab_body(s8, carry):
                t0 = pl.multiple_of(s8 * TSLAB, TSLAB)
                pltpu.sync_copy(x_sp.at[pl.ds(t0, TSLAB)], xs_v)
                pv16 = pos_v[pl.ds(t0, 16)]
                for tt in range(TSLAB):
                    pos_s = pv16[tt]

                    @plsc.parallel_loop(0, B // 16, unroll=4)
                    def _(vv):
                        idx16 = xs_v[tt, pl.ds(vv * 16, 16)]
                        g = plsc.load_gather(row_v, [idx16])
                        out_v[tt, pl.ds(vv * 16, 16)] = g + pos_s

                pltpu.sync_copy(out_v, out_hbm.at[pl.ds(t0, TSLAB), j])
                return carry

            lax.fori_loop(0, T // TSLAB, slab_body, 0)

    return k


def kernel(x, token_table, pos_table):
    out_t = _make_kernel()(token_table.T, x.T.astype(jnp.int32),
                           pos_table.T)
    return jnp.transpose(out_t, (2, 0, 1))


# speedup vs baseline: 6.3258x; 1.5800x over previous
import functools

import jax
import jax.numpy as jnp
from jax import lax
from jax.experimental import pallas as pl
from jax.experimental.pallas import tpu as pltpu
from jax.experimental.pallas import tpu_sc as plsc

D = 64
T = 200
B = 1024
V = 100000
NC = 2
NS = 16
NW = NC * NS

TSLAB = 8
NSLAB = T // TSLAB
NPAIR = (NSLAB - 1) // 2


@functools.lru_cache(maxsize=None)
def _make_kernel():
    mesh = plsc.VectorSubcoreMesh(core_axis_name="c", subcore_axis_name="s")

    @functools.partial(
        pl.kernel,
        mesh=mesh,
        compiler_params=pltpu.CompilerParams(
            use_tc_tiling_on_sc=False, needs_layout_passes=False),
        out_type=jax.ShapeDtypeStruct((T, 8, 8, 8, 128), jnp.float32),
        scratch_types=[
            pltpu.VMEM((V,), jnp.float32),
            pltpu.VMEM((2, TSLAB, 8, 128), jnp.float32),
            pltpu.VMEM((T + 16,), jnp.float32),
            pltpu.VMEM_SHARED((T, 8, 128), jnp.float32),
            pltpu.SemaphoreType.DMA,
            pltpu.SemaphoreType.DMA,
            pltpu.SemaphoreType.DMA,
            pltpu.SemaphoreType.DMA,
        ],
    )
    def k(tab_hbm, xt_hbm, pos_hbm, out_hbm, row_v, buf_v, pos_v, x_sp,
          sx0, sx1, so0, so1):
        cid = lax.axis_index("c")
        sid = lax.axis_index("s")
        wid = sid * NC + cid
        semx = (sx0, sx1)
        semo = (so0, so1)

        @pl.when(sid == 0)
        def _():
            pltpu.sync_copy(xt_hbm, x_sp)
        plsc.subcore_barrier()

        def start_x(s8, r):
            t0 = pl.multiple_of(s8 * TSLAB, TSLAB)
            pltpu.async_copy(x_sp.at[pl.ds(t0, TSLAB)], buf_v.at[r], semx[r])

        def wait_x(r):
            pltpu.make_async_copy(x_sp.at[pl.ds(0, TSLAB)], buf_v.at[r],
                                  semx[r]).wait()

        def start_out(s8, r, j):
            t0 = pl.multiple_of(s8 * TSLAB, TSLAB)
            pltpu.async_copy(
                buf_v.at[r],
                out_hbm.at[pl.ds(t0, TSLAB), j // 8, :, j % 8],
                semo[r])

        def wait_out(r, j):
            pltpu.make_async_copy(
                buf_v.at[r],
                out_hbm.at[pl.ds(0, TSLAB), j // 8, :, j % 8],
                semo[r]).wait()

        def compute(s8, r, j):
            t0 = pl.multiple_of(s8 * TSLAB, TSLAB)
            pv16 = pos_v[pl.ds(t0, 16)]
            for tt in range(TSLAB):
                pos_s = pv16[tt]

                @plsc.parallel_loop(0, B // 16, unroll=8)
                def _(vv):
                    cc = vv // 8
                    c0 = (vv % 8) * 16
                    idx16 = plsc.bitcast(buf_v[r, tt, cc, pl.ds(c0, 16)],
                                         jnp.int32)
                    g = plsc.load_gather(row_v, [idx16])
                    buf_v[r, tt, cc, pl.ds(c0, 16)] = g + pos_s

        for p in range(D // NW):
            j = p * NW + wid
            pltpu.sync_copy(tab_hbm.at[j], row_v)
            pltpu.sync_copy(pos_hbm.at[j], pos_v.at[pl.ds(0, T)])
            start_x(0, 0)

            def pair_body(g, carry):
                s8 = g * 2
                wait_x(0)

                @pl.when(g > 0)
                def _():
                    wait_out(1, j)
                start_x(s8 + 1, 1)
                compute(s8, 0, j)
                start_out(s8, 0, j)
                wait_x(1)
                wait_out(0, j)
                start_x(s8 + 2, 0)
                compute(s8 + 1, 1, j)
                start_out(s8 + 1, 1, j)
                return carry

            lax.fori_loop(0, NPAIR, pair_body, 0)
            wait_x(0)
            wait_out(1, j)
            compute(NSLAB - 1, 0, j)
            start_out(NSLAB - 1, 0, j)
            wait_out(0, j)

    return k


def kernel(x, token_table, pos_table):
    xt = jax.lax.bitcast_convert_type(
        x.T.astype(jnp.int32).reshape(T, 8, 128), jnp.float32)
    out5 = _make_kernel()(token_table.T, xt, pos_table.T)
    out_t = jnp.transpose(out5, (0, 1, 3, 2, 4)).reshape(T, D, B)
    return jnp.transpose(out_t, (2, 0, 1))
